# baseline (speedup 1.0000x reference)
import functools

import jax
import jax.numpy as jnp
from jax import lax
from jax.experimental import pallas as pl
from jax.experimental.pallas import tpu as pltpu
from jax.experimental.pallas import tpu_sc as plsc

N_NODES = 10000
N_EDGES = 320000
IN_DIM = 128
HID = 64
OUT_DIM = 32

NC = 2
NS = 16
NW = NC * NS
LANES = 16
BLK = 128
BLKS_PER_W = 81
EDGES_PER_W = BLK * BLKS_PER_W
E_PAD = EDGES_PER_W * NW
ROW_STRIDE = 624
ROW_SPAN = 640

_mesh = plsc.VectorSubcoreMesh(core_axis_name="c", subcore_axis_name="s")
_sc_params = pltpu.CompilerParams(needs_layout_passes=False,
                                  use_tc_tiling_on_sc=False)


def _zero_rows(rows_v, width):
    def zb(i, _):
        for q in range(width // LANES):
            rows_v[i, pl.ds(q * LANES, LANES)] = jnp.zeros((LANES,), jnp.float32)
        return 0
    lax.fori_loop(0, BLK, zb, 0)


def _zero_spmem_slice(rows_v, acc_s, sid):
    row0 = pl.multiple_of(sid * ROW_STRIDE, 8)
    for j in range(ROW_SPAN // BLK):
        pltpu.sync_copy(rows_v, acc_s.at[pl.ds(row0 + j * BLK, BLK)])


def _scale_rows(rows_v, g_v, goff, width):
    @plsc.parallel_loop(0, BLK, step=1, unroll=4)
    def sb(i):
        gb = plsc.load_gather(
            g_v, [jnp.broadcast_to(goff + i, (LANES,)).astype(jnp.int32)])
        for q in range(width // LANES):
            sl = pl.ds(q * LANES, LANES)
            rows_v[i, sl] = rows_v[i, sl] * gb


def _drain_acc(acc_s, acc_hbm, cid, sid):
    sl = pl.ds(pl.multiple_of(sid * ROW_STRIDE, 8), ROW_SPAN)
    pltpu.sync_copy(acc_s.at[sl], acc_hbm.at[cid].at[sl])


NBUF = 3


def _gather_scale_scatter(table_hbm, src2d_v, dst2d_v, g_v, rows, gsems, ssems,
                          acc_s, width, extra=None):
    n_outer = BLKS_PER_W // NBUF
    if extra is not None:
        for j in range(NBUF - 1):
            extra(j)
    for j in range(NBUF - 1):
        pltpu.async_copy(table_hbm.at[src2d_v.at[j]], rows[j], gsems[j])

    def outer(o, _):
        for k in range(NBUF):
            b = o * NBUF + k
            kp = (k + NBUF - 1) % NBUF
            pltpu.make_async_copy(table_hbm.at[src2d_v.at[b]],
                                  rows[k], gsems[k]).wait()
            _scale_rows(rows[k], g_v, b * BLK, width)
            pltpu.async_copy(rows[k], acc_s.at[dst2d_v.at[b]],
                             ssems[k], add=True)

            def tail():
                if extra is not None:
                    extra(b + NBUF - 1)
                @pl.when((k > 0) | (o >= 1))
                def _():
                    pltpu.make_async_copy(rows[kp], acc_s.at[dst2d_v.at[b]],
                                          ssems[kp]).wait()
                pltpu.async_copy(table_hbm.at[src2d_v.at[b + NBUF - 1]],
                                 rows[kp], gsems[kp])
            if k == 0:
                tail()
            else:
                pl.when(o < n_outer - 1)(tail)
        return 0

    lax.fori_loop(0, n_outer, outer, 0)

    for k in range(NBUF):
        pltpu.make_async_copy(rows[k], acc_s.at[dst2d_v.at[0]],
                              ssems[k]).wait()


@functools.partial(
    pl.kernel,
    out_type=[
        jax.ShapeDtypeStruct((E_PAD,), jnp.float32),
        jax.ShapeDtypeStruct((NW * N_NODES,), jnp.float32),
        jax.ShapeDtypeStruct((NC, N_NODES, HID), jnp.float32),
    ],
    mesh=_mesh,
    compiler_params=_sc_params,
    scratch_types=[
        pltpu.VMEM((N_NODES,), jnp.float32),
        pltpu.VMEM((N_NODES,), jnp.float32),
        pltpu.VMEM((N_NODES,), jnp.float32),
        pltpu.VMEM((BLKS_PER_W, BLK), jnp.int32),
        pltpu.VMEM((BLKS_PER_W, BLK), jnp.int32),
        pltpu.VMEM((EDGES_PER_W,), jnp.float32),
        [pltpu.VMEM((BLK, HID), jnp.float32)] * 3,
        pltpu.VMEM_SHARED((N_NODES, HID), jnp.float32),
        [pltpu.SemaphoreType.DMA] * 3,
        [pltpu.SemaphoreType.DMA] * 3,
    ],
)
def _sc_pass1(src_hbm, dst_hbm, asrc_hbm, adst_hbm, xs1_hbm,
              g_hbm, denom_hbm, acc_hbm,
              asrc_v, adst_v, denom_v, src2d_v, dst2d_v, g_v,
              rows, acc_s, gsems, ssems):
    cid = lax.axis_index("c")
    sid = lax.axis_index("s")
    wid = sid * NC + cid
    base = wid * EDGES_PER_W
    brow = pl.multiple_of(wid * BLKS_PER_W, 8)

    cps = [pltpu.async_copy(asrc_hbm, asrc_v, gsems[0]),
           pltpu.async_copy(adst_hbm, adst_v, gsems[1]),
           pltpu.async_copy(src_hbm.at[pl.ds(brow, BLKS_PER_W)], src2d_v,
                            ssems[0]),
           pltpu.async_copy(dst_hbm.at[pl.ds(brow, BLKS_PER_W)], dst2d_v,
                            ssems[1])]

    @plsc.parallel_loop(0, N_NODES // LANES, step=1, unroll=4)
    def zd(i):
        denom_v[pl.ds(pl.multiple_of(i * LANES, LANES), LANES)] = (
            jnp.zeros((LANES,), jnp.float32))
    for cp in cps:
        cp.wait()

    _zero_rows(rows[0], HID)
    _zero_spmem_slice(rows[0], acc_s, sid)
    plsc.subcore_barrier()

    def g_block(b):
        for q in range(BLK // LANES):
            s16 = src2d_v[b, pl.ds(q * LANES, LANES)]
            d16 = dst2d_v[b, pl.ds(q * LANES, LANES)]
            x = (plsc.load_gather(asrc_v, [s16])
                 + plsc.load_gather(adst_v, [d16]))
            g = jnp.exp(1.0 / (1.0 + jnp.exp(-x)))
            eid = base + b * BLK + q * LANES + lax.iota(jnp.int32, LANES)
            g = jnp.where(eid < N_EDGES, g, 0.0)
            g_v[pl.ds(pl.multiple_of(b * BLK + q * LANES, LANES), LANES)] = g
            plsc.addupdate_scatter(denom_v, [d16], g)

    _gather_scale_scatter(xs1_hbm, src2d_v, dst2d_v, g_v,
                          rows, gsems, ssems, acc_s, HID, extra=g_block)

    g_cp = pltpu.async_copy(
        g_v, g_hbm.at[pl.ds(pl.multiple_of(base, 8), EDGES_PER_W)], gsems[0])
    pltpu.sync_copy(
        denom_v,
        denom_hbm.at[pl.ds(pl.multiple_of(wid * N_NODES, 8), N_NODES)])
    plsc.subcore_barrier()
    _drain_acc(acc_s, acc_hbm, cid, sid)
    g_cp.wait()


@functools.partial(
    pl.kernel,
    out_type=[
        jax.ShapeDtypeStruct((NC, N_NODES, OUT_DIM), jnp.float32),
    ],
    mesh=_mesh,
    compiler_params=_sc_params,
    scratch_types=[
        pltpu.VMEM((BLKS_PER_W, BLK), jnp.int32),
        pltpu.VMEM((BLKS_PER_W, BLK), jnp.int32),
        pltpu.VMEM((EDGES_PER_W,), jnp.float32),
        [pltpu.VMEM((BLK, OUT_DIM), jnp.float32)] * 3,
        pltpu.VMEM_SHARED((N_NODES, OUT_DIM), jnp.float32),
        [pltpu.SemaphoreType.DMA] * 3,
        [pltpu.SemaphoreType.DMA] * 3,
    ],
)
def _sc_pass2(src_hbm, dst_hbm, g_hbm, h2_hbm,
              acc_hbm,
              src2d_v, dst2d_v, g_v, rows, acc_s, gsems, ssems):
    cid = lax.axis_index("c")
    sid = lax.axis_index("s")
    wid = sid * NC + cid
    base = wid * EDGES_PER_W
    brow = pl.multiple_of(wid * BLKS_PER_W, 8)

    cps = [pltpu.async_copy(src_hbm.at[pl.ds(brow, BLKS_PER_W)], src2d_v,
                            gsems[0]),
           pltpu.async_copy(dst_hbm.at[pl.ds(brow, BLKS_PER_W)], dst2d_v,
                            gsems[1]),
           pltpu.async_copy(g_hbm.at[pl.ds(pl.multiple_of(base, 8),
                                           EDGES_PER_W)], g_v, ssems[0])]
    for cp in cps:
        cp.wait()

    _zero_rows(rows[0], OUT_DIM)
    _zero_spmem_slice(rows[0], acc_s, sid)
    plsc.subcore_barrier()

    _gather_scale_scatter(h2_hbm, src2d_v, dst2d_v, g_v,
                          rows, gsems, ssems, acc_s, OUT_DIM)

    plsc.subcore_barrier()
    _drain_acc(acc_s, acc_hbm, cid, sid)


def _tc_a_body(feat, w1s, w1d, atts, attd, ei, w2s,
               xs1_o, asrc_o, adst_o, srcp_o, dstp_o, w2st_o, w1st_o):
    f = feat[...]
    xs1 = jnp.dot(f, w1s[...], preferred_element_type=jnp.float32)
    xs1_o[...] = xs1
    asrc_o[...] = jnp.sum(xs1 * atts[...], axis=1, keepdims=True)
    xd1 = jnp.dot(f, w1d[...], preferred_element_type=jnp.float32)
    adst_o[...] = jnp.sum(xd1 * attd[...], axis=1, keepdims=True)
    pad = (jnp.arange(E_PAD - N_EDGES, dtype=jnp.int32) % N_NODES)
    e = ei[...]
    srcp_o[...] = jnp.concatenate([e[0], pad]).reshape(NW * BLKS_PER_W, BLK)
    dstp_o[...] = jnp.concatenate([e[1], pad]).reshape(NW * BLKS_PER_W, BLK)
    w2st_o[...] = w2s[...].T
    w1st_o[...] = w1s[...].T


def _tc_b_body(acc, den, w2s, h2_o, rinv_o):
    a = acc[...]
    a = a[0] + a[1]
    d = jnp.sum(den[...], axis=0)
    rinv = 1.0 / (d + 1e-16)
    h1 = a * rinv[:, None]
    h1 = jnp.where(h1 > 0, h1, jnp.exp(h1) - 1.0)
    h2_o[...] = jnp.dot(h1, w2s[...], preferred_element_type=jnp.float32)
    rinv_o[...] = rinv[:, None]


def _tc_c_body(acc3, rinv, w2st, w1st, h4_o):
    a = acc3[...]
    x = jnp.dot(a[0] + a[1], w2st[...], preferred_element_type=jnp.float32)
    h3 = x * rinv[...]
    h3 = jnp.where(h3 > 0, h3, jnp.exp(h3) - 1.0)
    h4_o[...] = jnp.dot(h3, w1st[...], preferred_element_type=jnp.float32)


def kernel(features, edge_index, W1_s, W1_d, att1_s, att1_d, W2_s, W2_d):
    xs1, asrc, adst, srcp, dstp, w2st, w1st = pl.pallas_call(
        _tc_a_body,
        out_shape=[
            jax.ShapeDtypeStruct((N_NODES, HID), jnp.float32),
            jax.ShapeDtypeStruct((N_NODES, 1), jnp.float32),
            jax.ShapeDtypeStruct((N_NODES, 1), jnp.float32),
            jax.ShapeDtypeStruct((NW * BLKS_PER_W, BLK), jnp.int32),
            jax.ShapeDtypeStruct((NW * BLKS_PER_W, BLK), jnp.int32),
            jax.ShapeDtypeStruct((OUT_DIM, HID), jnp.float32),
            jax.ShapeDtypeStruct((HID, IN_DIM), jnp.float32),
        ],
    )(features, W1_s, W1_d, att1_s.reshape(1, HID), att1_d.reshape(1, HID),
      edge_index, W2_s)

    g, denom, acc1 = _sc_pass1(srcp, dstp, asrc.reshape(-1), adst.reshape(-1),
                               xs1)
    denom = denom.reshape(NW, N_NODES)

    h2, rinv = pl.pallas_call(
        _tc_b_body,
        out_shape=[
            jax.ShapeDtypeStruct((N_NODES, OUT_DIM), jnp.float32),
            jax.ShapeDtypeStruct((N_NODES, 1), jnp.float32),
        ],
    )(acc1, denom, W2_s)

    (acc3,) = _sc_pass2(srcp, dstp, g, h2)

    h4 = pl.pallas_call(
        _tc_c_body,
        out_shape=jax.ShapeDtypeStruct((N_NODES, IN_DIM), jnp.float32),
    )(acc3, rinv, w2st, w1st)

    return h2, h4


# speedup vs baseline: 1.0258x; 1.0258x over previous
import functools

import jax
import jax.numpy as jnp
from jax import lax
from jax.experimental import pallas as pl
from jax.experimental.pallas import tpu as pltpu
from jax.experimental.pallas import tpu_sc as plsc

N_NODES = 10000
N_EDGES = 320000
IN_DIM = 128
HID = 64
OUT_DIM = 32

NC = 2
NS = 16
NW = NC * NS
LANES = 16
BLK = 128
BLKS_PER_W = 81
EDGES_PER_W = BLK * BLKS_PER_W
E_PAD = EDGES_PER_W * NW
ROW_STRIDE = 624
ROW_SPAN = 640

_mesh = plsc.VectorSubcoreMesh(core_axis_name="c", subcore_axis_name="s")
_sc_params = pltpu.CompilerParams(needs_layout_passes=False,
                                  use_tc_tiling_on_sc=False)


def _zero_rows(rows_v, width):
    def zb(i, _):
        for q in range(width // LANES):
            rows_v[i, pl.ds(q * LANES, LANES)] = jnp.zeros((LANES,), jnp.float32)
        return 0
    lax.fori_loop(0, BLK, zb, 0)


def _zero_spmem_slice(rows_v, acc_s, sid):
    row0 = pl.multiple_of(sid * ROW_STRIDE, 8)
    for j in range(ROW_SPAN // BLK):
        pltpu.sync_copy(rows_v, acc_s.at[pl.ds(row0 + j * BLK, BLK)])


def _scale_rows(rows_v, g_v, goff, width):
    @plsc.parallel_loop(0, BLK, step=1, unroll=4)
    def sb(i):
        gb = plsc.load_gather(
            g_v, [jnp.broadcast_to(goff + i, (LANES,)).astype(jnp.int32)])
        for q in range(width // LANES):
            sl = pl.ds(q * LANES, LANES)
            rows_v[i, sl] = rows_v[i, sl] * gb


def _drain_acc(acc_s, acc_hbm, cid, sid):
    sl = pl.ds(pl.multiple_of(sid * ROW_STRIDE, 8), ROW_SPAN)
    pltpu.sync_copy(acc_s.at[sl], acc_hbm.at[cid].at[sl])


NBUF = 3


def _gather_scale_scatter(table_hbm, src2d_v, dst2d_v, g_v, rows, gsems, ssems,
                          acc_s, width, extra=None):
    n_outer = BLKS_PER_W // NBUF
    if extra is not None:
        for j in range(NBUF - 1):
            extra(j)
    for j in range(NBUF - 1):
        pltpu.async_copy(table_hbm.at[src2d_v.at[j]], rows[j], gsems[j])

    def outer(o, _):
        for k in range(NBUF):
            b = o * NBUF + k
            kp = (k + NBUF - 1) % NBUF
            pltpu.make_async_copy(table_hbm.at[src2d_v.at[b]],
                                  rows[k], gsems[k]).wait()
            _scale_rows(rows[k], g_v, b * BLK, width)
            pltpu.async_copy(rows[k], acc_s.at[dst2d_v.at[b]],
                             ssems[k], add=True)

            def tail():
                if extra is not None:
                    extra(b + NBUF - 1)
                @pl.when((k > 0) | (o >= 1))
                def _():
                    pltpu.make_async_copy(rows[kp], acc_s.at[dst2d_v.at[b]],
                                          ssems[kp]).wait()
                pltpu.async_copy(table_hbm.at[src2d_v.at[b + NBUF - 1]],
                                 rows[kp], gsems[kp])
            if k == 0:
                tail()
            else:
                pl.when(o < n_outer - 1)(tail)
        return 0

    lax.fori_loop(0, n_outer, outer, 0)

    for k in range(NBUF):
        pltpu.make_async_copy(rows[k], acc_s.at[dst2d_v.at[0]],
                              ssems[k]).wait()


@functools.partial(
    pl.kernel,
    out_type=[
        jax.ShapeDtypeStruct((E_PAD,), jnp.float32),
        jax.ShapeDtypeStruct((NW * N_NODES,), jnp.float32),
        jax.ShapeDtypeStruct((NC, N_NODES, HID), jnp.float32),
    ],
    mesh=_mesh,
    compiler_params=_sc_params,
    scratch_types=[
        pltpu.VMEM((N_NODES,), jnp.float32),
        pltpu.VMEM((N_NODES,), jnp.float32),
        pltpu.VMEM((N_NODES,), jnp.float32),
        pltpu.VMEM((BLKS_PER_W, BLK), jnp.int32),
        pltpu.VMEM((BLKS_PER_W, BLK), jnp.int32),
        pltpu.VMEM((EDGES_PER_W,), jnp.float32),
        [pltpu.VMEM((BLK, HID), jnp.float32)] * 3,
        pltpu.VMEM_SHARED((N_NODES, HID), jnp.float32),
        [pltpu.SemaphoreType.DMA] * 3,
        [pltpu.SemaphoreType.DMA] * 3,
    ],
)
def _sc_pass1(src_hbm, dst_hbm, asrc_hbm, adst_hbm, xs1_hbm,
              g_hbm, denom_hbm, acc_hbm,
              asrc_v, adst_v, denom_v, src2d_v, dst2d_v, g_v,
              rows, acc_s, gsems, ssems):
    cid = lax.axis_index("c")
    sid = lax.axis_index("s")
    wid = sid * NC + cid
    base = wid * EDGES_PER_W
    brow = pl.multiple_of(wid * BLKS_PER_W, 8)

    cps = [pltpu.async_copy(asrc_hbm, asrc_v, gsems[0]),
           pltpu.async_copy(adst_hbm, adst_v, gsems[1]),
           pltpu.async_copy(src_hbm.at[pl.ds(brow, BLKS_PER_W)], src2d_v,
                            ssems[0]),
           pltpu.async_copy(dst_hbm.at[pl.ds(brow, BLKS_PER_W)], dst2d_v,
                            ssems[1])]

    @plsc.parallel_loop(0, N_NODES // LANES, step=1, unroll=4)
    def zd(i):
        denom_v[pl.ds(pl.multiple_of(i * LANES, LANES), LANES)] = (
            jnp.zeros((LANES,), jnp.float32))
    for cp in cps:
        cp.wait()

    _zero_rows(rows[0], HID)
    _zero_spmem_slice(rows[0], acc_s, sid)
    plsc.subcore_barrier()

    @plsc.parallel_loop(0, BLKS_PER_W, step=1)
    def p1(b):
        for q in range(BLK // LANES):
            s16 = src2d_v[b, pl.ds(q * LANES, LANES)]
            d16 = dst2d_v[b, pl.ds(q * LANES, LANES)]
            x = (plsc.load_gather(asrc_v, [s16])
                 + plsc.load_gather(adst_v, [d16]))
            g = jnp.exp(1.0 / (1.0 + jnp.exp(-x)))
            eid = base + b * BLK + q * LANES + lax.iota(jnp.int32, LANES)
            g = jnp.where(eid < N_EDGES, g, 0.0)
            g_v[pl.ds(pl.multiple_of(b * BLK + q * LANES, LANES), LANES)] = g
            plsc.addupdate_scatter(denom_v, [d16], g)

    _gather_scale_scatter(xs1_hbm, src2d_v, dst2d_v, g_v,
                          rows, gsems, ssems, acc_s, HID)

    g_cp = pltpu.async_copy(
        g_v, g_hbm.at[pl.ds(pl.multiple_of(base, 8), EDGES_PER_W)], gsems[0])
    pltpu.sync_copy(
        denom_v,
        denom_hbm.at[pl.ds(pl.multiple_of(wid * N_NODES, 8), N_NODES)])
    plsc.subcore_barrier()
    _drain_acc(acc_s, acc_hbm, cid, sid)
    g_cp.wait()


@functools.partial(
    pl.kernel,
    out_type=[
        jax.ShapeDtypeStruct((NC, N_NODES, OUT_DIM), jnp.float32),
    ],
    mesh=_mesh,
    compiler_params=_sc_params,
    scratch_types=[
        pltpu.VMEM((BLKS_PER_W, BLK), jnp.int32),
        pltpu.VMEM((BLKS_PER_W, BLK), jnp.int32),
        pltpu.VMEM((EDGES_PER_W,), jnp.float32),
        [pltpu.VMEM((BLK, OUT_DIM), jnp.float32)] * 3,
        pltpu.VMEM_SHARED((N_NODES, OUT_DIM), jnp.float32),
        [pltpu.SemaphoreType.DMA] * 3,
        [pltpu.SemaphoreType.DMA] * 3,
    ],
)
def _sc_pass2(src_hbm, dst_hbm, g_hbm, h2_hbm,
              acc_hbm,
              src2d_v, dst2d_v, g_v, rows, acc_s, gsems, ssems):
    cid = lax.axis_index("c")
    sid = lax.axis_index("s")
    wid = sid * NC + cid
    base = wid * EDGES_PER_W
    brow = pl.multiple_of(wid * BLKS_PER_W, 8)

    cps = [pltpu.async_copy(src_hbm.at[pl.ds(brow, BLKS_PER_W)], src2d_v,
                            gsems[0]),
           pltpu.async_copy(dst_hbm.at[pl.ds(brow, BLKS_PER_W)], dst2d_v,
                            gsems[1]),
           pltpu.async_copy(g_hbm.at[pl.ds(pl.multiple_of(base, 8),
                                           EDGES_PER_W)], g_v, ssems[0])]
    for cp in cps:
        cp.wait()

    _zero_rows(rows[0], OUT_DIM)
    _zero_spmem_slice(rows[0], acc_s, sid)
    plsc.subcore_barrier()

    _gather_scale_scatter(h2_hbm, src2d_v, dst2d_v, g_v,
                          rows, gsems, ssems, acc_s, OUT_DIM)

    plsc.subcore_barrier()
    _drain_acc(acc_s, acc_hbm, cid, sid)


def _tc_a_body(feat, w1s, w1d, atts, attd, ei, w2s,
               xs1_o, asrc_o, adst_o, srcp_o, dstp_o, w2st_o, w1st_o):
    f = feat[...]
    xs1 = jnp.dot(f, w1s[...], preferred_element_type=jnp.float32)
    xs1_o[...] = xs1
    asrc_o[...] = jnp.sum(xs1 * atts[...], axis=1, keepdims=True)
    xd1 = jnp.dot(f, w1d[...], preferred_element_type=jnp.float32)
    adst_o[...] = jnp.sum(xd1 * attd[...], axis=1, keepdims=True)
    pad = (jnp.arange(E_PAD - N_EDGES, dtype=jnp.int32) % N_NODES)
    e = ei[...]
    srcp_o[...] = jnp.concatenate([e[0], pad]).reshape(NW * BLKS_PER_W, BLK)
    dstp_o[...] = jnp.concatenate([e[1], pad]).reshape(NW * BLKS_PER_W, BLK)
    w2st_o[...] = w2s[...].T
    w1st_o[...] = w1s[...].T


def _tc_b_body(acc, den, w2s, h2_o, rinv_o):
    a = acc[...]
    a = a[0] + a[1]
    d = jnp.sum(den[...], axis=0)
    rinv = 1.0 / (d + 1e-16)
    h1 = a * rinv[:, None]
    h1 = jnp.where(h1 > 0, h1, jnp.exp(h1) - 1.0)
    h2_o[...] = jnp.dot(h1, w2s[...], preferred_element_type=jnp.float32)
    rinv_o[...] = rinv[:, None]


def _tc_c_body(acc3, rinv, w2st, w1st, h4_o):
    a = acc3[...]
    x = jnp.dot(a[0] + a[1], w2st[...], preferred_element_type=jnp.float32)
    h3 = x * rinv[...]
    h3 = jnp.where(h3 > 0, h3, jnp.exp(h3) - 1.0)
    h4_o[...] = jnp.dot(h3, w1st[...], preferred_element_type=jnp.float32)


def kernel(features, edge_index, W1_s, W1_d, att1_s, att1_d, W2_s, W2_d):
    xs1, asrc, adst, srcp, dstp, w2st, w1st = pl.pallas_call(
        _tc_a_body,
        out_shape=[
            jax.ShapeDtypeStruct((N_NODES, HID), jnp.float32),
            jax.ShapeDtypeStruct((N_NODES, 1), jnp.float32),
            jax.ShapeDtypeStruct((N_NODES, 1), jnp.float32),
            jax.ShapeDtypeStruct((NW * BLKS_PER_W, BLK), jnp.int32),
            jax.ShapeDtypeStruct((NW * BLKS_PER_W, BLK), jnp.int32),
            jax.ShapeDtypeStruct((OUT_DIM, HID), jnp.float32),
            jax.ShapeDtypeStruct((HID, IN_DIM), jnp.float32),
        ],
    )(features, W1_s, W1_d, att1_s.reshape(1, HID), att1_d.reshape(1, HID),
      edge_index, W2_s)

    g, denom, acc1 = _sc_pass1(srcp, dstp, asrc.reshape(-1), adst.reshape(-1),
                               xs1)
    denom = denom.reshape(NW, N_NODES)

    h2, rinv = pl.pallas_call(
        _tc_b_body,
        out_shape=[
            jax.ShapeDtypeStruct((N_NODES, OUT_DIM), jnp.float32),
            jax.ShapeDtypeStruct((N_NODES, 1), jnp.float32),
        ],
    )(acc1, denom, W2_s)

    (acc3,) = _sc_pass2(srcp, dstp, g, h2)

    h4 = pl.pallas_call(
        _tc_c_body,
        out_shape=jax.ShapeDtypeStruct((N_NODES, IN_DIM), jnp.float32),
    )(acc3, rinv, w2st, w1st)

    return h2, h4
